# baseline (speedup 1.0000x reference)
import functools

import jax
import jax.numpy as jnp
from jax import lax
from jax.experimental import pallas as pl
from jax.experimental.pallas import tpu as pltpu
from jax.experimental.pallas import tpu_sc as plsc

NM, ND, NA = 20000, 10000, 20000
IN, HID, H, OUT = 512, 256, 8, 3
D = HID // H

NMP = NM + 96
NTILES = 16
NW = 32
RT = NMP // NTILES
CA = 128
CB = 128
CG = 32


def _mm2_body(x_ref, w1_ref, b1_ref, w2a_ref, b2a_ref, w2b_ref, b2b_ref,
              oa_ref, ob_ref):
    h = jnp.dot(x_ref[...], w1_ref[...], preferred_element_type=jnp.float32)
    h = h + b1_ref[...]
    oa_ref[...] = jnp.dot(h, w2a_ref[...],
                          preferred_element_type=jnp.float32) + b2a_ref[...]
    ob_ref[...] = jnp.dot(h, w2b_ref[...],
                          preferred_element_type=jnp.float32) + b2b_ref[...]


def _proj_kv(x, w1, b1, w2a, b2a, w2b, b2b, bn=1000):
    n = x.shape[0]
    return pl.pallas_call(
        _mm2_body,
        grid=(n // bn,),
        in_specs=[
            pl.BlockSpec((bn, x.shape[1]), lambda i: (i, 0)),
            pl.BlockSpec((x.shape[1], HID), lambda i: (0, 0)),
            pl.BlockSpec((HID,), lambda i: (0,)),
            pl.BlockSpec((HID, HID), lambda i: (0, 0)),
            pl.BlockSpec((HID,), lambda i: (0,)),
            pl.BlockSpec((HID, HID), lambda i: (0, 0)),
            pl.BlockSpec((HID,), lambda i: (0,)),
        ],
        out_specs=[
            pl.BlockSpec((bn, HID), lambda i: (i, 0)),
            pl.BlockSpec((bn, HID), lambda i: (i, 0)),
        ],
        out_shape=[
            jax.ShapeDtypeStruct((n, HID), jnp.float32),
            jax.ShapeDtypeStruct((n, HID), jnp.float32),
        ],
    )(x, w1, b1, w2a, b2a, w2b, b2b)


def _mmh_body(x_ref, w1_ref, b1_ref, w2_ref, b2_ref, h_ref, o_ref):
    h = jnp.dot(x_ref[...], w1_ref[...], preferred_element_type=jnp.float32)
    h = h + b1_ref[...]
    h_ref[...] = h
    o_ref[...] = jnp.dot(h, w2_ref[...],
                         preferred_element_type=jnp.float32) + b2_ref[...]


def _proj_hq(x, w1, b1, w2, b2, bn=1000):
    n = x.shape[0]
    return pl.pallas_call(
        _mmh_body,
        grid=(n // bn,),
        in_specs=[
            pl.BlockSpec((bn, x.shape[1]), lambda i: (i, 0)),
            pl.BlockSpec((x.shape[1], HID), lambda i: (0, 0)),
            pl.BlockSpec((HID,), lambda i: (0,)),
            pl.BlockSpec((HID, HID), lambda i: (0, 0)),
            pl.BlockSpec((HID,), lambda i: (0,)),
        ],
        out_specs=[
            pl.BlockSpec((bn, HID), lambda i: (i, 0)),
            pl.BlockSpec((bn, HID), lambda i: (i, 0)),
        ],
        out_shape=[
            jax.ShapeDtypeStruct((n, HID), jnp.float32),
            jax.ShapeDtypeStruct((n, HID), jnp.float32),
        ],
    )(x, w1, b1, w2, b2)


def _gather_body(ew, q_hbm, kt_hbm, vt_hbm, src_hbm, dst_hbm,
                 qe_hbm, ke_hbm, ve_hbm,
                 srcb0, srcb1, dstb0, dstb1, qr0, qr1, kr0, kr1, vr0, vr1,
                 sq0, sq1, sk0, sk1, sv0, sv1,
                 oq0, oq1, ok0, ok1, ov0, ov1):
    cid = lax.axis_index("c")
    sid = lax.axis_index("s")
    base = (sid * 2 + cid) * ew
    nC = ew // CG
    srcb = (srcb0, srcb1)
    dstb = (dstb0, dstb1)
    qr = (qr0, qr1)
    kr = (kr0, kr1)
    vr = (vr0, vr1)
    sq = (sq0, sq1)
    sk = (sk0, sk1)
    sv = (sv0, sv1)
    oq = (oq0, oq1)
    ok = (ok0, ok1)
    ov = (ov0, ov1)

    def load_and_gather(i, b):
        off = base + i * CG
        pltpu.sync_copy(src_hbm.at[pl.ds(off, CG)], srcb[b])
        pltpu.sync_copy(dst_hbm.at[pl.ds(off, CG)], dstb[b])
        pltpu.async_copy(q_hbm.at[dstb[b]], qr[b], sq[b])
        pltpu.async_copy(kt_hbm.at[srcb[b]], kr[b], sk[b])
        pltpu.async_copy(vt_hbm.at[srcb[b]], vr[b], sv[b])

    def step(j, _):
        i = j * 2
        for b in range(2):
            off = base + (i + b) * CG
            pltpu.make_async_copy(q_hbm.at[dstb[b]], qr[b], sq[b]).wait()
            pltpu.make_async_copy(kt_hbm.at[srcb[b]], kr[b], sk[b]).wait()
            pltpu.make_async_copy(vt_hbm.at[srcb[b]], vr[b], sv[b]).wait()
            pltpu.async_copy(qr[b], qe_hbm.at[pl.ds(off, CG)], oq[b])
            pltpu.async_copy(kr[b], ke_hbm.at[pl.ds(off, CG)], ok[b])
            pltpu.async_copy(vr[b], ve_hbm.at[pl.ds(off, CG)], ov[b])
            pltpu.make_async_copy(qr[b], qe_hbm.at[pl.ds(off, CG)], oq[b]).wait()
            pltpu.make_async_copy(kr[b], ke_hbm.at[pl.ds(off, CG)], ok[b]).wait()
            pltpu.make_async_copy(vr[b], ve_hbm.at[pl.ds(off, CG)], ov[b]).wait()
            nxt = i + b + 2

            @pl.when(nxt < nC)
            def _():
                load_and_gather(nxt, b)
        return 0

    for b in range(2):
        load_and_gather(b, b)
    lax.fori_loop(0, nC // 2, step, 0)


def _gather_sc(q_pad, kt, vt, src_p, dst_p, epad):
    ew = epad // NW
    mesh = plsc.VectorSubcoreMesh(core_axis_name="c", subcore_axis_name="s")
    f = pl.kernel(
        functools.partial(_gather_body, ew),
        out_type=[
            jax.ShapeDtypeStruct((epad, 256), jnp.float32),
            jax.ShapeDtypeStruct((epad, 256), jnp.float32),
            jax.ShapeDtypeStruct((epad, 256), jnp.float32),
        ],
        mesh=mesh,
        compiler_params=pltpu.CompilerParams(use_tc_tiling_on_sc=False,
                                             needs_layout_passes=False),
        scratch_types=(
            [pltpu.VMEM((CG,), jnp.int32) for _ in range(4)]
            + [pltpu.VMEM((CG, 256), jnp.float32) for _ in range(6)]
            + [pltpu.SemaphoreType.DMA for _ in range(12)]
        ),
    )
    return f(q_pad, kt, vt, src_p, dst_p)


def _alpha_body(qe_ref, ke_ref, ve_ref, e32_ref, *we_ref):
    cols8 = lax.broadcasted_iota(jnp.int32, (HID, 8), 0)
    rows8 = lax.broadcasted_iota(jnp.int32, (HID, 8), 1)
    sel = (cols8 // D == rows8).astype(jnp.float32)
    p = qe_ref[...] * ke_ref[...]
    r = jnp.dot(p, sel, preferred_element_type=jnp.float32)
    e = jnp.exp(r)
    cols32 = lax.broadcasted_iota(jnp.int32, (8, 32), 1)
    rows32 = lax.broadcasted_iota(jnp.int32, (8, 32), 0)
    emb = (cols32 == rows32).astype(jnp.float32)
    e32_ref[...] = jnp.dot(e, emb, preferred_element_type=jnp.float32)
    ve = ve_ref[...]
    for h in range(8):
        we_ref[h][...] = ve[:, h * 32:(h + 1) * 32] * e[:, h][:, None]


def _alpha_tc(qe, ke, ve, be=2048):
    epad = qe.shape[0]
    return pl.pallas_call(
        _alpha_body,
        grid=(epad // be,),
        in_specs=[
            pl.BlockSpec((be, HID), lambda i: (i, 0)),
            pl.BlockSpec((be, HID), lambda i: (i, 0)),
            pl.BlockSpec((be, HID), lambda i: (i, 0)),
        ],
        out_specs=[pl.BlockSpec((be, 32), lambda i: (i, 0))
                   for _ in range(9)],
        out_shape=[jax.ShapeDtypeStruct((epad, 32), jnp.float32)
                   for _ in range(9)],
    )(qe, ke, ve)


def _scatter_body(ew, e32_hbm, w0, w1, w2, w3, w4, w5, w6, w7,
                  dst_hbm, zacc_hbm, acc_hbm, s_hbm,
                  dstb0, dstb1, wr0, wr1, ss0, ss1, acc_sp):
    cid = lax.axis_index("c")
    sid = lax.axis_index("s")
    base = (sid * 2 + cid) * ew
    rt0 = sid * RT
    nB = ew // CB
    dstb = (dstb0, dstb1)
    wr = (wr0, wr1)
    ss = (ss0, ss1)

    def zero_acc():
        for j in range(4):
            pltpu.sync_copy(zacc_hbm, acc_sp.at[pl.ds(rt0 + j * 256, 256)])
        pltpu.sync_copy(zacc_hbm.at[pl.ds(0, RT - 1024)],
                        acc_sp.at[pl.ds(rt0 + 1024, RT - 1024)])

    def one_pass(load_rows):
        def step(j, _):
            i = j * 2
            for b in range(2):
                ci = i + b
                off = base + ci * CB
                pltpu.sync_copy(dst_hbm.at[pl.ds(off, CB)], dstb[b])
                load_rows(off, wr[b])
                pltpu.sync_copy(wr[b], acc_sp.at[dstb[b]], add=True)
            return 0
        lax.fori_loop(0, nB // 2, step, 0)
        if nB % 2 == 1:
            off = base + (nB - 1) * CB
            pltpu.sync_copy(dst_hbm.at[pl.ds(off, CB)], dstb[0])
            load_rows(off, wr[0])
            pltpu.sync_copy(wr[0], acc_sp.at[dstb[0]], add=True)

    zero_acc()
    plsc.subcore_barrier()

    def mk_load(table):
        def load(off, dst_ref):
            pltpu.sync_copy(table.at[pl.ds(off, CB)], dst_ref)
        return load
    one_pass(mk_load(e32_hbm))
    plsc.subcore_barrier()
    pltpu.sync_copy(acc_sp.at[pl.ds(rt0, RT)], s_hbm.at[cid, pl.ds(rt0, RT)])
    plsc.subcore_barrier()

    for hh, wt in enumerate((w0, w1, w2, w3, w4, w5, w6, w7)):
        zero_acc()
        plsc.subcore_barrier()
        one_pass(mk_load(wt))
        plsc.subcore_barrier()
        pltpu.sync_copy(acc_sp.at[pl.ds(rt0, RT)],
                        acc_hbm.at[cid, hh, pl.ds(rt0, RT)])
        plsc.subcore_barrier()


def _scatter_sc(planes, dst_p, epad):
    ew = epad // NW
    zacc = jnp.zeros((256, 32), jnp.float32)
    mesh = plsc.VectorSubcoreMesh(core_axis_name="c", subcore_axis_name="s")
    f = pl.kernel(
        functools.partial(_scatter_body, ew),
        out_type=[
            jax.ShapeDtypeStruct((2, 8, NMP, 32), jnp.float32),
            jax.ShapeDtypeStruct((2, NMP, 32), jnp.float32),
        ],
        mesh=mesh,
        compiler_params=pltpu.CompilerParams(use_tc_tiling_on_sc=False,
                                             needs_layout_passes=False),
        scratch_types=[
            pltpu.VMEM((CB,), jnp.int32),
            pltpu.VMEM((CB,), jnp.int32),
            pltpu.VMEM((CB, 32), jnp.float32),
            pltpu.VMEM((CB, 32), jnp.float32),
            pltpu.SemaphoreType.DMA,
            pltpu.SemaphoreType.DMA,
            pltpu.VMEM_SHARED((NMP, 32), jnp.float32),
        ],
    )
    return f(*planes, dst_p, zacc)


def _edge_sc(q_pad, kt, vt, src, dst, epad):
    e = src.shape[0]
    srcp = jnp.concatenate([src, jnp.zeros((epad - e,), jnp.int32)])
    dstp = jnp.concatenate([dst, jnp.full((epad - e,), NM, jnp.int32)])
    qe, ke, ve = _gather_sc(q_pad, kt, vt, srcp, dstp, epad)
    planes = _alpha_tc(qe, ke, ve)
    return _scatter_sc(planes, dstp, epad)


def _final_body(accdm_ref, sdm_ref, accam_ref, sam_ref, h0_ref, ow_ref,
                ob_ref, lw_ref, lb_ref, beta_ref, out_ref):
    cols = lax.broadcasted_iota(jnp.int32, (8, HID), 1)
    rows = lax.broadcasted_iota(jnp.int32, (8, HID), 0)
    expander = (cols // D == rows).astype(jnp.float32)

    def norm(acc_ref, s_ref):
        ab = acc_ref[0] + acc_ref[1]
        a = jnp.concatenate([ab[j] for j in range(8)], axis=1)
        s = s_ref[0, :, :8] + s_ref[1, :, :8]
        rinv = 1.0 / (s + 1e-16)
        rrep = jnp.dot(rinv, expander, preferred_element_type=jnp.float32)
        return a * rrep

    acc = norm(accdm_ref, sdm_ref) + norm(accam_ref, sam_ref)
    g = jax.nn.gelu(acc)
    o = jnp.dot(g, ow_ref[...], preferred_element_type=jnp.float32) + ob_ref[...]
    beta = beta_ref[0]
    mixed = beta * o + (1.0 - beta) * h0_ref[...]
    out_ref[...] = jnp.dot(mixed, lw_ref[...],
                           preferred_element_type=jnp.float32) + lb_ref[...]


def _final(accdm, sdm, accam, sam, h0, outw, outb, linw_pad, linb_pad, beta,
           bn=1000):
    n = h0.shape[0]
    op = linw_pad.shape[1]
    return pl.pallas_call(
        _final_body,
        grid=(n // bn,),
        in_specs=[
            pl.BlockSpec((2, 8, bn, 32), lambda i: (0, 0, i, 0)),
            pl.BlockSpec((2, bn, 32), lambda i: (0, i, 0)),
            pl.BlockSpec((2, 8, bn, 32), lambda i: (0, 0, i, 0)),
            pl.BlockSpec((2, bn, 32), lambda i: (0, i, 0)),
            pl.BlockSpec((bn, HID), lambda i: (i, 0)),
            pl.BlockSpec((HID, HID), lambda i: (0, 0)),
            pl.BlockSpec((HID,), lambda i: (0,)),
            pl.BlockSpec((HID, op), lambda i: (0, 0)),
            pl.BlockSpec((op,), lambda i: (0,)),
            pl.BlockSpec(memory_space=pltpu.SMEM),
        ],
        out_specs=pl.BlockSpec((bn, op), lambda i: (i, 0)),
        out_shape=jax.ShapeDtypeStruct((n, op), jnp.float32),
    )(accdm, sdm, accam, sam, h0, outw, outb, linw_pad, linb_pad, beta)


def _fold_kv(kW, kB, rel, scale):
    w = jnp.einsum('chd,hde->che', kW.reshape(HID, H, D), rel)
    b = jnp.einsum('hd,hde->he', kB.reshape(H, D), rel)
    if scale is not None:
        w = w * scale[None, :, None]
        b = b * scale[:, None]
    return w.reshape(HID, HID), b.reshape(HID)


def kernel(x_movie, x_director, x_actor, edge_index_md, edge_index_dm,
           edge_index_ma, edge_index_am, preW, preB, kW, kB, qW, qB, vW, vB,
           aRel, mRel, pRel, outW, outB, skip, linW, linB):
    scale_dm = pRel[1] / (D ** 0.5)
    scale_am = pRel[3] / (D ** 0.5)
    kw1, kb1 = _fold_kv(kW[1], kB[1], aRel[1], scale_dm)
    vw1, vb1 = _fold_kv(vW[1], vB[1], mRel[1], None)
    kw2, kb2 = _fold_kv(kW[2], kB[2], aRel[3], scale_am)
    vw2, vb2 = _fold_kv(vW[2], vB[2], mRel[3], None)

    h0, q0 = _proj_hq(x_movie, preW[0], preB[0], qW[0], qB[0])
    k1t, v1t = _proj_kv(x_director, preW[1], preB[1], kw1, kb1, vw1, vb1)
    k2t, v2t = _proj_kv(x_actor, preW[2], preB[2], kw2, kb2, vw2, vb2)

    q_pad = jnp.pad(q0, ((0, NMP - NM), (0, 0)))
    accdm, sdm = _edge_sc(q_pad, k1t, v1t, edge_index_dm[0],
                          edge_index_dm[1], 102400)
    accam, sam = _edge_sc(q_pad, k2t, v2t, edge_index_am[0],
                          edge_index_am[1], 200704)

    beta = jax.nn.sigmoid(skip[0:1])
    linw_pad = jnp.pad(linW, ((0, 0), (0, 128 - OUT)))
    linb_pad = jnp.pad(linB, ((0, 128 - OUT),))
    out_pad = _final(accdm, sdm, accam, sam, h0, outW[0], outB[0],
                     linw_pad, linb_pad, beta)
    return out_pad[:, :OUT]


# speedup vs baseline: 1.0224x; 1.0224x over previous
import functools

import jax
import jax.numpy as jnp
from jax import lax
from jax.experimental import pallas as pl
from jax.experimental.pallas import tpu as pltpu
from jax.experimental.pallas import tpu_sc as plsc

NM, ND, NA = 20000, 10000, 20000
IN, HID, H, OUT = 512, 256, 8, 3
D = HID // H

NMP = NM + 96
NTILES = 16
NW = 32
RT = NMP // NTILES
CA = 128
CB = 128
CG = 32


def _mm2_body(x_ref, w1_ref, b1_ref, w2a_ref, b2a_ref, w2b_ref, b2b_ref,
              oa_ref, ob_ref):
    h = jnp.dot(x_ref[...], w1_ref[...], preferred_element_type=jnp.float32)
    h = h + b1_ref[...]
    oa_ref[...] = jnp.dot(h, w2a_ref[...],
                          preferred_element_type=jnp.float32) + b2a_ref[...]
    ob_ref[...] = jnp.dot(h, w2b_ref[...],
                          preferred_element_type=jnp.float32) + b2b_ref[...]


def _proj_kv(x, w1, b1, w2a, b2a, w2b, b2b, bn=1000):
    n = x.shape[0]
    return pl.pallas_call(
        _mm2_body,
        grid=(n // bn,),
        in_specs=[
            pl.BlockSpec((bn, x.shape[1]), lambda i: (i, 0)),
            pl.BlockSpec((x.shape[1], HID), lambda i: (0, 0)),
            pl.BlockSpec((HID,), lambda i: (0,)),
            pl.BlockSpec((HID, HID), lambda i: (0, 0)),
            pl.BlockSpec((HID,), lambda i: (0,)),
            pl.BlockSpec((HID, HID), lambda i: (0, 0)),
            pl.BlockSpec((HID,), lambda i: (0,)),
        ],
        out_specs=[
            pl.BlockSpec((bn, HID), lambda i: (i, 0)),
            pl.BlockSpec((bn, HID), lambda i: (i, 0)),
        ],
        out_shape=[
            jax.ShapeDtypeStruct((n, HID), jnp.float32),
            jax.ShapeDtypeStruct((n, HID), jnp.float32),
        ],
    )(x, w1, b1, w2a, b2a, w2b, b2b)


def _mmh_body(x_ref, w1_ref, b1_ref, w2_ref, b2_ref, h_ref, o_ref):
    h = jnp.dot(x_ref[...], w1_ref[...], preferred_element_type=jnp.float32)
    h = h + b1_ref[...]
    h_ref[...] = h
    o_ref[...] = jnp.dot(h, w2_ref[...],
                         preferred_element_type=jnp.float32) + b2_ref[...]


def _proj_hq(x, w1, b1, w2, b2, bn=1000):
    n = x.shape[0]
    return pl.pallas_call(
        _mmh_body,
        grid=(n // bn,),
        in_specs=[
            pl.BlockSpec((bn, x.shape[1]), lambda i: (i, 0)),
            pl.BlockSpec((x.shape[1], HID), lambda i: (0, 0)),
            pl.BlockSpec((HID,), lambda i: (0,)),
            pl.BlockSpec((HID, HID), lambda i: (0, 0)),
            pl.BlockSpec((HID,), lambda i: (0,)),
        ],
        out_specs=[
            pl.BlockSpec((bn, HID), lambda i: (i, 0)),
            pl.BlockSpec((bn, HID), lambda i: (i, 0)),
        ],
        out_shape=[
            jax.ShapeDtypeStruct((n, HID), jnp.float32),
            jax.ShapeDtypeStruct((n, HID), jnp.float32),
        ],
    )(x, w1, b1, w2, b2)


def _gather_body(ew, q_hbm, kt_hbm, vt_hbm, src_hbm, dst_hbm,
                 qe_hbm, ke_hbm, ve_hbm,
                 srcb0, srcb1, dstb0, dstb1, qr0, qr1, kr0, kr1, vr0, vr1,
                 sq0, sq1, sk0, sk1, sv0, sv1,
                 oq0, oq1, ok0, ok1, ov0, ov1):
    cid = lax.axis_index("c")
    sid = lax.axis_index("s")
    base = (sid * 2 + cid) * ew
    nC = ew // CG
    srcb = (srcb0, srcb1)
    dstb = (dstb0, dstb1)
    qr = (qr0, qr1)
    kr = (kr0, kr1)
    vr = (vr0, vr1)
    sq = (sq0, sq1)
    sk = (sk0, sk1)
    sv = (sv0, sv1)
    oq = (oq0, oq1)
    ok = (ok0, ok1)
    ov = (ov0, ov1)

    def load_and_gather(i, b):
        off = base + i * CG
        pltpu.sync_copy(src_hbm.at[pl.ds(off, CG)], srcb[b])
        pltpu.sync_copy(dst_hbm.at[pl.ds(off, CG)], dstb[b])
        pltpu.async_copy(q_hbm.at[dstb[b]], qr[b], sq[b])
        pltpu.async_copy(kt_hbm.at[srcb[b]], kr[b], sk[b])
        pltpu.async_copy(vt_hbm.at[srcb[b]], vr[b], sv[b])

    def step(j, _):
        i = j * 2
        for b in range(2):
            off = base + (i + b) * CG
            pltpu.make_async_copy(q_hbm.at[dstb[b]], qr[b], sq[b]).wait()
            pltpu.make_async_copy(kt_hbm.at[srcb[b]], kr[b], sk[b]).wait()
            pltpu.make_async_copy(vt_hbm.at[srcb[b]], vr[b], sv[b]).wait()
            pltpu.async_copy(qr[b], qe_hbm.at[pl.ds(off, CG)], oq[b])
            pltpu.async_copy(kr[b], ke_hbm.at[pl.ds(off, CG)], ok[b])
            pltpu.async_copy(vr[b], ve_hbm.at[pl.ds(off, CG)], ov[b])
            pltpu.make_async_copy(qr[b], qe_hbm.at[pl.ds(off, CG)], oq[b]).wait()
            pltpu.make_async_copy(kr[b], ke_hbm.at[pl.ds(off, CG)], ok[b]).wait()
            pltpu.make_async_copy(vr[b], ve_hbm.at[pl.ds(off, CG)], ov[b]).wait()
            nxt = i + b + 2

            @pl.when(nxt < nC)
            def _():
                load_and_gather(nxt, b)
        return 0

    for b in range(2):
        load_and_gather(b, b)
    lax.fori_loop(0, nC // 2, step, 0)


def _gather_sc(q_pad, kt, vt, src_p, dst_p, epad):
    ew = epad // NW
    mesh = plsc.VectorSubcoreMesh(core_axis_name="c", subcore_axis_name="s")
    f = pl.kernel(
        functools.partial(_gather_body, ew),
        out_type=[
            jax.ShapeDtypeStruct((epad, 256), jnp.float32),
            jax.ShapeDtypeStruct((epad, 256), jnp.float32),
            jax.ShapeDtypeStruct((epad, 256), jnp.float32),
        ],
        mesh=mesh,
        compiler_params=pltpu.CompilerParams(use_tc_tiling_on_sc=False,
                                             needs_layout_passes=False),
        scratch_types=(
            [pltpu.VMEM((CG,), jnp.int32) for _ in range(4)]
            + [pltpu.VMEM((CG, 256), jnp.float32) for _ in range(6)]
            + [pltpu.SemaphoreType.DMA for _ in range(12)]
        ),
    )
    return f(q_pad, kt, vt, src_p, dst_p)


def _alpha_body(qe_ref, ke_ref, ve_ref, e32_ref, *we_ref):
    cols8 = lax.broadcasted_iota(jnp.int32, (HID, 8), 0)
    rows8 = lax.broadcasted_iota(jnp.int32, (HID, 8), 1)
    sel = (cols8 // D == rows8).astype(jnp.float32)
    p = qe_ref[...] * ke_ref[...]
    r = jnp.dot(p, sel, preferred_element_type=jnp.float32)
    e = jnp.exp(r)
    cols32 = lax.broadcasted_iota(jnp.int32, (8, 32), 1)
    rows32 = lax.broadcasted_iota(jnp.int32, (8, 32), 0)
    emb = (cols32 == rows32).astype(jnp.float32)
    e32_ref[...] = jnp.dot(e, emb, preferred_element_type=jnp.float32)
    ve = ve_ref[...]
    for h in range(8):
        we_ref[h][...] = ve[:, h * 32:(h + 1) * 32] * e[:, h][:, None]


def _alpha_tc(qe, ke, ve, be=2048):
    epad = qe.shape[0]
    return pl.pallas_call(
        _alpha_body,
        grid=(epad // be,),
        in_specs=[
            pl.BlockSpec((be, HID), lambda i: (i, 0)),
            pl.BlockSpec((be, HID), lambda i: (i, 0)),
            pl.BlockSpec((be, HID), lambda i: (i, 0)),
        ],
        out_specs=[pl.BlockSpec((be, 32), lambda i: (i, 0))
                   for _ in range(9)],
        out_shape=[jax.ShapeDtypeStruct((epad, 32), jnp.float32)
                   for _ in range(9)],
    )(qe, ke, ve)


def _scatter_body(ew, e32_hbm, w0, w1, w2, w3, w4, w5, w6, w7,
                  dst_hbm, zacc_hbm, acc_hbm, s_hbm,
                  dstb0, dstb1, wr0, wr1, ss0, ss1, acc_sp):
    cid = lax.axis_index("c")
    sid = lax.axis_index("s")
    base = (sid * 2 + cid) * ew
    rt0 = sid * RT
    nB = ew // CB
    dstb = (dstb0, dstb1)
    wr = (wr0, wr1)
    ss = (ss0, ss1)

    def zero_acc():
        for j in range(4):
            pltpu.sync_copy(zacc_hbm, acc_sp.at[pl.ds(rt0 + j * 256, 256)])
        pltpu.sync_copy(zacc_hbm.at[pl.ds(0, RT - 1024)],
                        acc_sp.at[pl.ds(rt0 + 1024, RT - 1024)])

    def one_pass(load_rows):
        def chunk(ci, b):
            off = base + ci * CB
            pltpu.sync_copy(dst_hbm.at[pl.ds(off, CB)], dstb[b])
            load_rows(off, wr[b])
            pltpu.async_copy(wr[b], acc_sp.at[dstb[b]], ss[b], add=True)

        def step(j, _):
            i = j * 2
            for b in range(2):
                ci = i + b

                @pl.when(ci >= 2)
                def _():
                    pltpu.make_async_copy(
                        wr[b], acc_sp.at[dstb[b]], ss[b]).wait()
                chunk(ci, b)
            return 0
        lax.fori_loop(0, nB // 2, step, 0)
        tail = nB % 2
        if tail:
            pltpu.make_async_copy(wr[0], acc_sp.at[dstb[0]], ss[0]).wait()
            chunk(nB - 1, 0)
        for b in range(2):
            pltpu.make_async_copy(wr[b], acc_sp.at[dstb[b]], ss[b]).wait()
        if nB > 2 - tail:
            pass

    zero_acc()
    plsc.subcore_barrier()

    def mk_load(table):
        def load(off, dst_ref):
            pltpu.sync_copy(table.at[pl.ds(off, CB)], dst_ref)
        return load
    one_pass(mk_load(e32_hbm))
    plsc.subcore_barrier()
    pltpu.sync_copy(acc_sp.at[pl.ds(rt0, RT)], s_hbm.at[cid, pl.ds(rt0, RT)])
    plsc.subcore_barrier()

    for hh, wt in enumerate((w0, w1, w2, w3, w4, w5, w6, w7)):
        zero_acc()
        plsc.subcore_barrier()
        one_pass(mk_load(wt))
        plsc.subcore_barrier()
        pltpu.sync_copy(acc_sp.at[pl.ds(rt0, RT)],
                        acc_hbm.at[cid, hh, pl.ds(rt0, RT)])
        plsc.subcore_barrier()


def _scatter_sc(planes, dst_p, epad):
    ew = epad // NW
    zacc = jnp.zeros((256, 32), jnp.float32)
    mesh = plsc.VectorSubcoreMesh(core_axis_name="c", subcore_axis_name="s")
    f = pl.kernel(
        functools.partial(_scatter_body, ew),
        out_type=[
            jax.ShapeDtypeStruct((2, 8, NMP, 32), jnp.float32),
            jax.ShapeDtypeStruct((2, NMP, 32), jnp.float32),
        ],
        mesh=mesh,
        compiler_params=pltpu.CompilerParams(use_tc_tiling_on_sc=False,
                                             needs_layout_passes=False),
        scratch_types=[
            pltpu.VMEM((CB,), jnp.int32),
            pltpu.VMEM((CB,), jnp.int32),
            pltpu.VMEM((CB, 32), jnp.float32),
            pltpu.VMEM((CB, 32), jnp.float32),
            pltpu.SemaphoreType.DMA,
            pltpu.SemaphoreType.DMA,
            pltpu.VMEM_SHARED((NMP, 32), jnp.float32),
        ],
    )
    return f(*planes, dst_p, zacc)


def _edge_sc(q_pad, kt, vt, src, dst, epad):
    e = src.shape[0]
    srcp = jnp.concatenate([src, jnp.zeros((epad - e,), jnp.int32)])
    dstp = jnp.concatenate([dst, jnp.full((epad - e,), NM, jnp.int32)])
    qe, ke, ve = _gather_sc(q_pad, kt, vt, srcp, dstp, epad)
    planes = _alpha_tc(qe, ke, ve)
    return _scatter_sc(planes, dstp, epad)


def _final_body(accdm_ref, sdm_ref, accam_ref, sam_ref, h0_ref, ow_ref,
                ob_ref, lw_ref, lb_ref, beta_ref, out_ref):
    cols = lax.broadcasted_iota(jnp.int32, (8, HID), 1)
    rows = lax.broadcasted_iota(jnp.int32, (8, HID), 0)
    expander = (cols // D == rows).astype(jnp.float32)

    def norm(acc_ref, s_ref):
        ab = acc_ref[0] + acc_ref[1]
        a = jnp.concatenate([ab[j] for j in range(8)], axis=1)
        s = s_ref[0, :, :8] + s_ref[1, :, :8]
        rinv = 1.0 / (s + 1e-16)
        rrep = jnp.dot(rinv, expander, preferred_element_type=jnp.float32)
        return a * rrep

    acc = norm(accdm_ref, sdm_ref) + norm(accam_ref, sam_ref)
    g = jax.nn.gelu(acc)
    o = jnp.dot(g, ow_ref[...], preferred_element_type=jnp.float32) + ob_ref[...]
    beta = beta_ref[0]
    mixed = beta * o + (1.0 - beta) * h0_ref[...]
    out_ref[...] = jnp.dot(mixed, lw_ref[...],
                           preferred_element_type=jnp.float32) + lb_ref[...]


def _final(accdm, sdm, accam, sam, h0, outw, outb, linw_pad, linb_pad, beta,
           bn=1000):
    n = h0.shape[0]
    op = linw_pad.shape[1]
    return pl.pallas_call(
        _final_body,
        grid=(n // bn,),
        in_specs=[
            pl.BlockSpec((2, 8, bn, 32), lambda i: (0, 0, i, 0)),
            pl.BlockSpec((2, bn, 32), lambda i: (0, i, 0)),
            pl.BlockSpec((2, 8, bn, 32), lambda i: (0, 0, i, 0)),
            pl.BlockSpec((2, bn, 32), lambda i: (0, i, 0)),
            pl.BlockSpec((bn, HID), lambda i: (i, 0)),
            pl.BlockSpec((HID, HID), lambda i: (0, 0)),
            pl.BlockSpec((HID,), lambda i: (0,)),
            pl.BlockSpec((HID, op), lambda i: (0, 0)),
            pl.BlockSpec((op,), lambda i: (0,)),
            pl.BlockSpec(memory_space=pltpu.SMEM),
        ],
        out_specs=pl.BlockSpec((bn, op), lambda i: (i, 0)),
        out_shape=jax.ShapeDtypeStruct((n, op), jnp.float32),
    )(accdm, sdm, accam, sam, h0, outw, outb, linw_pad, linb_pad, beta)


def _fold_kv(kW, kB, rel, scale):
    w = jnp.einsum('chd,hde->che', kW.reshape(HID, H, D), rel)
    b = jnp.einsum('hd,hde->he', kB.reshape(H, D), rel)
    if scale is not None:
        w = w * scale[None, :, None]
        b = b * scale[:, None]
    return w.reshape(HID, HID), b.reshape(HID)


def kernel(x_movie, x_director, x_actor, edge_index_md, edge_index_dm,
           edge_index_ma, edge_index_am, preW, preB, kW, kB, qW, qB, vW, vB,
           aRel, mRel, pRel, outW, outB, skip, linW, linB):
    scale_dm = pRel[1] / (D ** 0.5)
    scale_am = pRel[3] / (D ** 0.5)
    kw1, kb1 = _fold_kv(kW[1], kB[1], aRel[1], scale_dm)
    vw1, vb1 = _fold_kv(vW[1], vB[1], mRel[1], None)
    kw2, kb2 = _fold_kv(kW[2], kB[2], aRel[3], scale_am)
    vw2, vb2 = _fold_kv(vW[2], vB[2], mRel[3], None)

    h0, q0 = _proj_hq(x_movie, preW[0], preB[0], qW[0], qB[0])
    k1t, v1t = _proj_kv(x_director, preW[1], preB[1], kw1, kb1, vw1, vb1)
    k2t, v2t = _proj_kv(x_actor, preW[2], preB[2], kw2, kb2, vw2, vb2)

    q_pad = jnp.pad(q0, ((0, NMP - NM), (0, 0)))
    accdm, sdm = _edge_sc(q_pad, k1t, v1t, edge_index_dm[0],
                          edge_index_dm[1], 102400)
    accam, sam = _edge_sc(q_pad, k2t, v2t, edge_index_am[0],
                          edge_index_am[1], 200704)

    beta = jax.nn.sigmoid(skip[0:1])
    linw_pad = jnp.pad(linW, ((0, 0), (0, 128 - OUT)))
    linb_pad = jnp.pad(linB, ((0, 128 - OUT),))
    out_pad = _final(accdm, sdm, accam, sam, h0, outW[0], outB[0],
                     linw_pad, linb_pad, beta)
    return out_pad[:, :OUT]
